# Initial kernel scaffold; baseline (speedup 1.0000x reference)
import jax
import jax.numpy as jnp
from jax.experimental import pallas as pl
from jax.experimental.pallas import tpu as pltpu

EMBED_DIM = 1024
NUM_EXPERTS = 64
ACTIVE_EXPERTS = 2
HIDDEN = 4 * EMBED_DIM
NTOK = 8

BH = 1024
NH = HIDDEN // BH


def _router_body(x_ref, wr_ref, br_ref, i0_ref, i1_ref, w0_ref, w1_ref):
    scores = jnp.maximum(
        jnp.dot(x_ref[...], wr_ref[...], preferred_element_type=jnp.float32)
        + br_ref[...],
        0.0,
    )
    i0 = jnp.argmax(scores, axis=1)
    v0 = jnp.max(scores, axis=1)
    col = jax.lax.broadcasted_iota(jnp.int32, scores.shape, 1)
    masked = jnp.where(col == i0[:, None], -jnp.inf, scores)
    i1 = jnp.argmax(masked, axis=1)
    v1 = jnp.max(masked, axis=1)
    e1 = jnp.exp(v1 - v0)
    denom = 1.0 + e1
    i0_ref[...] = i0[:, None].astype(jnp.int32)
    i1_ref[...] = i1[:, None].astype(jnp.int32)
    w0_ref[...] = (1.0 / denom)[:, None]
    w1_ref[...] = (e1 / denom)[:, None]


def _expert_body(idx_ref, wts_ref, x_ref, w1_ref, b1_ref, w2_ref, b2_ref,
                 out_ref):
    p = pl.program_id(0)
    h = pl.program_id(1)
    w = wts_ref[p]

    hid = jnp.maximum(
        jnp.dot(x_ref[...], w1_ref[0], preferred_element_type=jnp.float32)
        + b1_ref[...],
        0.0,
    )
    part = jnp.dot(hid, w2_ref[0], preferred_element_type=jnp.float32)

    @pl.when(jnp.logical_and(h == 0, p % 2 == 0))
    def _init():
        out_ref[...] = jnp.zeros_like(out_ref)

    @pl.when(h == 0)
    def _bias():
        out_ref[...] += w * b2_ref[...]

    out_ref[...] += w * part


@jax.jit
def kernel(x, Wr, br, W1, b1, W2, b2):
    x2 = x.reshape(NTOK, EMBED_DIM)

    i0, i1, w0, w1 = pl.pallas_call(
        _router_body,
        out_shape=(
            jax.ShapeDtypeStruct((NTOK, 1), jnp.int32),
            jax.ShapeDtypeStruct((NTOK, 1), jnp.int32),
            jax.ShapeDtypeStruct((NTOK, 1), jnp.float32),
            jax.ShapeDtypeStruct((NTOK, 1), jnp.float32),
        ),
    )(x2, Wr, br.reshape(1, NUM_EXPERTS))

    idx = jnp.concatenate([i0, i1], axis=1).reshape(-1)
    wts = jnp.concatenate([w0, w1], axis=1).reshape(-1)

    grid_spec = pltpu.PrefetchScalarGridSpec(
        num_scalar_prefetch=2,
        grid=(2 * NTOK, NH),
        in_specs=[
            pl.BlockSpec((1, EMBED_DIM), lambda p, h, idx, wts: (p // 2, 0)),
            pl.BlockSpec((1, EMBED_DIM, BH), lambda p, h, idx, wts: (idx[p], 0, h)),
            pl.BlockSpec((1, BH), lambda p, h, idx, wts: (idx[p], h)),
            pl.BlockSpec((1, BH, EMBED_DIM), lambda p, h, idx, wts: (idx[p], h, 0)),
            pl.BlockSpec((1, EMBED_DIM), lambda p, h, idx, wts: (idx[p], 0)),
        ],
        out_specs=pl.BlockSpec((1, EMBED_DIM), lambda p, h, idx, wts: (p // 2, 0)),
    )

    out = pl.pallas_call(
        _expert_body,
        grid_spec=grid_spec,
        out_shape=jax.ShapeDtypeStruct((NTOK, EMBED_DIM), jnp.float32),
    )(idx, wts, x2, W1, b1, W2, b2)

    return out.reshape(x.shape)


# speedup vs baseline: 4.9923x; 4.9923x over previous
import jax
import jax.numpy as jnp
from jax.experimental import pallas as pl
from jax.experimental.pallas import tpu as pltpu

EMBED_DIM = 1024
NUM_EXPERTS = 64
ACTIVE_EXPERTS = 2
HIDDEN = 4 * EMBED_DIM
NTOK = 8

BH = 1024
NH = HIDDEN // BH


def _router_body(x_ref, wr_ref, br_ref, i0_ref, i1_ref, w0_ref, w1_ref):
    scores = jnp.maximum(
        jnp.dot(x_ref[...], wr_ref[...], preferred_element_type=jnp.float32)
        + br_ref[...],
        0.0,
    )
    i0 = jnp.argmax(scores, axis=1)
    v0 = jnp.max(scores, axis=1)
    col = jax.lax.broadcasted_iota(jnp.int32, scores.shape, 1)
    masked = jnp.where(col == i0[:, None], -jnp.inf, scores)
    i1 = jnp.argmax(masked, axis=1)
    v1 = jnp.max(masked, axis=1)
    e1 = jnp.exp(v1 - v0)
    denom = 1.0 + e1
    i0_ref[...] = i0[:, None].astype(jnp.int32)
    i1_ref[...] = i1[:, None].astype(jnp.int32)
    w0_ref[...] = (1.0 / denom)[:, None]
    w1_ref[...] = (e1 / denom)[:, None]


def _expert_body(idx_ref, wts_ref, x_ref, w1_ref, b1_ref, w2_ref, b2_ref,
                 out_ref):
    p = pl.program_id(0)
    h = pl.program_id(1)
    w = wts_ref[p]

    hid = jnp.maximum(
        jnp.dot(x_ref[0], w1_ref[0], preferred_element_type=jnp.float32)
        + b1_ref[0],
        0.0,
    )
    part = jnp.dot(hid, w2_ref[0], preferred_element_type=jnp.float32)

    @pl.when(jnp.logical_and(h == 0, p % 2 == 0))
    def _init():
        out_ref[...] = jnp.zeros_like(out_ref)

    @pl.when(h == 0)
    def _bias():
        out_ref[0] += w * b2_ref[0]

    out_ref[0] += w * part


@jax.jit
def kernel(x, Wr, br, W1, b1, W2, b2):
    x2 = x.reshape(NTOK, EMBED_DIM)

    i0, i1, w0, w1 = pl.pallas_call(
        _router_body,
        out_shape=(
            jax.ShapeDtypeStruct((NTOK, 1), jnp.int32),
            jax.ShapeDtypeStruct((NTOK, 1), jnp.int32),
            jax.ShapeDtypeStruct((NTOK, 1), jnp.float32),
            jax.ShapeDtypeStruct((NTOK, 1), jnp.float32),
        ),
    )(x2, Wr, br.reshape(1, NUM_EXPERTS))

    idx = jnp.concatenate([i0, i1], axis=1).reshape(-1)
    wts = jnp.concatenate([w0, w1], axis=1).reshape(-1)

    grid_spec = pltpu.PrefetchScalarGridSpec(
        num_scalar_prefetch=2,
        grid=(2 * NTOK, NH),
        in_specs=[
            pl.BlockSpec((1, 1, EMBED_DIM), lambda p, h, idx, wts: (p // 2, 0, 0)),
            pl.BlockSpec((1, EMBED_DIM, BH), lambda p, h, idx, wts: (idx[p], 0, h)),
            pl.BlockSpec((1, 1, BH), lambda p, h, idx, wts: (idx[p], 0, h)),
            pl.BlockSpec((1, BH, EMBED_DIM), lambda p, h, idx, wts: (idx[p], h, 0)),
            pl.BlockSpec((1, 1, EMBED_DIM), lambda p, h, idx, wts: (idx[p], 0, 0)),
        ],
        out_specs=pl.BlockSpec((1, 1, EMBED_DIM), lambda p, h, idx, wts: (p // 2, 0, 0)),
    )

    out = pl.pallas_call(
        _expert_body,
        grid_spec=grid_spec,
        out_shape=jax.ShapeDtypeStruct((NTOK, 1, EMBED_DIM), jnp.float32),
    )(idx, wts, x2.reshape(NTOK, 1, EMBED_DIM), W1,
      b1.reshape(NUM_EXPERTS, 1, HIDDEN), W2,
      b2.reshape(NUM_EXPERTS, 1, EMBED_DIM))

    return out.reshape(x.shape)
